# baseline (speedup 1.0000x reference)
import functools

import jax
import jax.numpy as jnp
from jax import lax
from jax.experimental import pallas as pl
from jax.experimental.pallas import tpu as pltpu
from jax.experimental.pallas import tpu_sc as plsc

NUM_CORES = 2
NUM_SUBCORES = 16
NW = NUM_CORES * NUM_SUBCORES

BATCH = 16384
HIST = 50
DIM = 64
TOTAL = BATCH * HIST
PER_W = TOTAL // NW
CHUNK = 512
NCHUNK = PER_W // CHUNK
G = 1
NGRP = NCHUNK // G
NSUP = NGRP // 2

_mesh = plsc.VectorSubcoreMesh(
    core_axis_name="c", subcore_axis_name="s",
    num_cores=NUM_CORES, num_subcores=NUM_SUBCORES,
)


@functools.partial(
    pl.kernel,
    out_type=jax.ShapeDtypeStruct((NW, NCHUNK, CHUNK, DIM), jnp.float32),
    mesh=_mesh,
    scratch_types=[
        pltpu.VMEM((PER_W,), jnp.int32),
        pltpu.VMEM((G, CHUNK, DIM), jnp.float32),
        pltpu.VMEM((G, CHUNK, DIM), jnp.float32),
        pltpu.SemaphoreType.DMA,
        pltpu.SemaphoreType.DMA,
        pltpu.SemaphoreType.DMA,
        pltpu.SemaphoreType.DMA,
    ],
    compiler_params=pltpu.CompilerParams(use_tc_tiling_on_sc=False),
)
def _embed_sc(idx_hbm, table_hbm, out_hbm, idx_v, buf0, buf1, g0, g1, o0, o1):
    wid = lax.axis_index("s") * NUM_CORES + lax.axis_index("c")
    pltpu.sync_copy(idx_hbm.at[wid], idx_v)

    def fire_gathers(grp, buf, sem):
        for b in range(G):
            j = grp * G + b
            pltpu.async_copy(
                table_hbm.at[idx_v.at[pl.ds(j * CHUNK, CHUNK)]], buf.at[b], sem)

    def wait_gathers(buf, sem):
        for b in range(G):
            pltpu.make_async_copy(
                table_hbm.at[pl.ds(0, CHUNK)], buf.at[b], sem).wait()

    def fire_writebacks(grp, buf, sem):
        for b in range(G):
            pltpu.async_copy(buf.at[b], out_hbm.at[wid, grp * G + b], sem)

    def wait_writebacks(buf, sem):
        for b in range(G):
            pltpu.make_async_copy(buf.at[b], out_hbm.at[wid, 0], sem).wait()

    fire_gathers(0, buf0, g0)

    def body(t, carry):
        @pl.when(t > 0)
        def _():
            wait_writebacks(buf1, o1)
        fire_gathers(2 * t + 1, buf1, g1)
        wait_gathers(buf0, g0)
        fire_writebacks(2 * t, buf0, o0)
        wait_writebacks(buf0, o0)
        @pl.when(t + 1 < NSUP)
        def _():
            fire_gathers(2 * t + 2, buf0, g0)
        wait_gathers(buf1, g1)
        fire_writebacks(2 * t + 1, buf1, o1)
        return carry

    lax.fori_loop(0, NSUP, body, 0)
    wait_writebacks(buf1, o1)


def kernel(input_ids, weight):
    idx = input_ids.reshape(NW, PER_W).astype(jnp.int32)
    out = _embed_sc(idx, weight)
    return out.reshape(BATCH, HIST, DIM)


# speedup vs baseline: 1.0001x; 1.0001x over previous
import functools

import jax
import jax.numpy as jnp
from jax import lax
from jax.experimental import pallas as pl
from jax.experimental.pallas import tpu as pltpu
from jax.experimental.pallas import tpu_sc as plsc

NUM_CORES = 2
NUM_SUBCORES = 16
NW = NUM_CORES * NUM_SUBCORES

BATCH = 16384
HIST = 50
DIM = 64
TOTAL = BATCH * HIST
PER_W = TOTAL // NW
CHUNK = 256
NCHUNK = PER_W // CHUNK
NBUF = 4
NOUT = NCHUNK // NBUF

_mesh = plsc.VectorSubcoreMesh(
    core_axis_name="c", subcore_axis_name="s",
    num_cores=NUM_CORES, num_subcores=NUM_SUBCORES,
)


@functools.partial(
    pl.kernel,
    out_type=jax.ShapeDtypeStruct((NW, NCHUNK, CHUNK, DIM), jnp.float32),
    mesh=_mesh,
    scratch_types=[
        pltpu.VMEM((PER_W,), jnp.int32),
        *[pltpu.VMEM((CHUNK, DIM), jnp.float32) for _ in range(NBUF)],
        *[pltpu.SemaphoreType.DMA for _ in range(NBUF)],
        *[pltpu.SemaphoreType.DMA for _ in range(NBUF)],
    ],
    compiler_params=pltpu.CompilerParams(use_tc_tiling_on_sc=False),
)
def _embed_sc(idx_hbm, table_hbm, out_hbm, idx_v, *scratch):
    bufs = scratch[:NBUF]
    gsem = scratch[NBUF:2 * NBUF]
    osem = scratch[2 * NBUF:]

    wid = lax.axis_index("s") * NUM_CORES + lax.axis_index("c")
    pltpu.sync_copy(idx_hbm.at[wid], idx_v)

    def fire_gather(j, buf, sem):
        pltpu.async_copy(
            table_hbm.at[idx_v.at[pl.ds(j * CHUNK, CHUNK)]], buf, sem)

    def wait_gather(buf, sem):
        pltpu.make_async_copy(
            table_hbm.at[pl.ds(0, CHUNK)], buf, sem).wait()

    def fire_writeback(j, buf, sem):
        pltpu.async_copy(buf, out_hbm.at[wid, j], sem)

    def wait_writeback(buf, sem):
        pltpu.make_async_copy(buf, out_hbm.at[wid, 0], sem).wait()

    for b in range(NBUF):
        fire_gather(b, bufs[b], gsem[b])

    def body(t, carry):
        j0 = t * NBUF
        for b in range(NBUF):
            j = j0 + b
            wait_gather(bufs[b], gsem[b])
            fire_writeback(j, bufs[b], osem[b])

            @pl.when(j + NBUF < NCHUNK)
            def _():
                wait_writeback(bufs[b], osem[b])
                fire_gather(j + NBUF, bufs[b], gsem[b])
        return carry

    lax.fori_loop(0, NOUT, body, 0)
    for b in range(NBUF):
        wait_writeback(bufs[b], osem[b])


def kernel(input_ids, weight):
    idx = input_ids.reshape(NW, PER_W).astype(jnp.int32)
    out = _embed_sc(idx, weight)
    return out.reshape(BATCH, HIST, DIM)
